# baseline (speedup 1.0000x reference)
import functools

import jax
import jax.numpy as jnp
from jax import lax
from jax.experimental import pallas as pl
from jax.experimental.pallas import tpu as pltpu
from jax.experimental.pallas import tpu_sc as plsc

N = 10000
E = 320000
G = 512
D_IN = 128
H = 32

NC = 2
NS = 16
NW = NC * NS
L = 16

KE = 80
EPAD = NW * KE * 128
NB = 8
AH = 4
NG = KE // NB
KN = 3
NPAD = NW * KN * 128

NP = 10240
NP_T = NP // NS
GP = 640
GP_T = GP // NS


def _fill_zeros(ref, nrows):
    z16 = jnp.zeros((L,), jnp.float32)
    w = ref.shape[1]

    def body(i, _):
        for j0 in range(0, w, L):
            ref[i, j0:j0 + L] = z16
        return 0

    lax.fori_loop(0, nrows, body, 0)


def _fill_ones(ref, nrows):
    o16 = jnp.ones((L,), jnp.float32)

    def body(i, _):
        ref[i, 0:L] = o16
        return 0

    lax.fori_loop(0, nrows, body, 0)


@functools.cache
def _sc_kernels():
    mesh = plsc.VectorSubcoreMesh(core_axis_name="c", subcore_axis_name="s")

    @functools.partial(
        pl.kernel,
        out_type=[
            jax.ShapeDtypeStruct((NC, NP, L), jnp.float32),
            jax.ShapeDtypeStruct((NC, GP, L), jnp.float32),
        ],
        mesh=mesh,
        compiler_params=pltpu.CompilerParams(use_tc_tiling_on_sc=False),
        scratch_types=[
            pltpu.VMEM((KE, 128), jnp.int32),
            pltpu.VMEM((KN, 128), jnp.int32),
            pltpu.VMEM((128, L), jnp.float32),
            pltpu.VMEM((128, L), jnp.float32),
            pltpu.VMEM_SHARED((NP, L), jnp.float32),
            pltpu.VMEM_SHARED((GP, L), jnp.float32),
            pltpu.SemaphoreType.DMA,
            pltpu.SemaphoreType.DMA,
        ],
    )
    def _deg_cnt(dst_hbm, bat_hbm, deg_out, cnt_out,
                 dstv, batv, ones_v, zero_v, deg_sh, cnt_sh, dsem, bsem):
        c = lax.axis_index("c")
        s = lax.axis_index("s")
        wid = s * NC + c
        pltpu.sync_copy(dst_hbm.at[wid], dstv)
        pltpu.sync_copy(bat_hbm.at[wid], batv)
        _fill_ones(ones_v, 128)
        _fill_zeros(zero_v, 128)
        nbase = s * NP_T
        for j in range(NP_T // 128):
            pltpu.sync_copy(zero_v, deg_sh.at[pl.ds(nbase + j * 128, 128)])
        gbase = s * GP_T
        pltpu.sync_copy(zero_v.at[pl.ds(0, GP_T)],
                        cnt_sh.at[pl.ds(gbase, GP_T)])
        plsc.subcore_barrier()

        def d_start(j):
            pltpu.async_copy(ones_v, deg_sh.at[dstv.at[j]], dsem, add=True)

        def d_wait():
            pltpu.make_async_copy(ones_v, deg_sh.at[dstv.at[0]], dsem).wait()

        for j in range(KN):
            pltpu.async_copy(ones_v, cnt_sh.at[batv.at[j]], bsem, add=True)
        for b in range(NB):
            d_start(b)

        def grp(g, _):
            for b in range(NB):
                d_start(g * NB + b)
                d_wait()
            return 0

        lax.fori_loop(1, NG, grp, 0)
        for b in range(NB):
            d_wait()
        for j in range(KN):
            pltpu.make_async_copy(ones_v, cnt_sh.at[batv.at[0]], bsem).wait()
        plsc.subcore_barrier()
        pltpu.sync_copy(deg_sh.at[pl.ds(nbase, NP_T)],
                        deg_out.at[c, pl.ds(nbase, NP_T)])
        pltpu.sync_copy(cnt_sh.at[pl.ds(gbase, GP_T)],
                        cnt_out.at[c, pl.ds(gbase, GP_T)])

    @functools.partial(
        pl.kernel,
        out_type=jax.ShapeDtypeStruct((NC, NP, H), jnp.float32),
        mesh=mesh,
        compiler_params=pltpu.CompilerParams(use_tc_tiling_on_sc=False),
        scratch_types=[
            pltpu.VMEM((KE, 128), jnp.int32),
            pltpu.VMEM((KE, 128), jnp.int32),
            pltpu.VMEM((NB, 128, H), jnp.float32),
            pltpu.VMEM((128, H), jnp.float32),
            pltpu.VMEM_SHARED((NP, H), jnp.float32),
            pltpu.SemaphoreType.DMA((NB,)),
            pltpu.SemaphoreType.DMA((NB,)),
        ],
    )
    def _agg(tab_hbm, src_hbm, dst_hbm, out_hbm,
             srcv, dstv, rows, zero_v, acc_sh, gsem, ssem):
        c = lax.axis_index("c")
        s = lax.axis_index("s")
        wid = s * NC + c
        pltpu.sync_copy(src_hbm.at[wid], srcv)
        pltpu.sync_copy(dst_hbm.at[wid], dstv)
        _fill_zeros(zero_v, 128)
        nbase = s * NP_T
        for j in range(NP_T // 128):
            pltpu.sync_copy(zero_v, acc_sh.at[pl.ds(nbase + j * 128, 128)])
        plsc.subcore_barrier()

        def g_start(j, b):
            pltpu.async_copy(tab_hbm.at[srcv.at[j]], rows.at[b], gsem.at[b])

        def g_wait(j, b):
            pltpu.make_async_copy(
                tab_hbm.at[srcv.at[j]], rows.at[b], gsem.at[b]).wait()

        def s_start(j, b):
            pltpu.async_copy(rows.at[b], acc_sh.at[dstv.at[j]], ssem.at[b],
                             add=True)

        def s_wait(j, b):
            pltpu.make_async_copy(
                rows.at[b], acc_sh.at[dstv.at[j]], ssem.at[b]).wait()

        for b in range(AH):
            g_start(b, b)
        for b in range(NB):
            j = b
            g_wait(j, b)
            s_start(j, b)
            bn = (b + AH) % NB
            if b >= AH:
                s_wait(j + AH - NB, bn)
            g_start(j + AH, bn)

        def grp(g, _):
            for b in range(NB):
                j = g * NB + b
                g_wait(j, b)
                s_start(j, b)
                bn = (b + AH) % NB
                s_wait(j + AH - NB, bn)
                g_start(j + AH, bn)
            return 0

        lax.fori_loop(1, NG - 1, grp, 0)
        for b in range(NB):
            j = (NG - 1) * NB + b
            g_wait(j, b)
            s_start(j, b)
            if b < AH:
                bn = (b + AH) % NB
                s_wait(j + AH - NB, bn)
                g_start(j + AH, bn)
        for b in range(NB):
            s_wait(KE - NB + b, b)
        plsc.subcore_barrier()
        pltpu.sync_copy(acc_sh.at[pl.ds(nbase, NP_T)],
                        out_hbm.at[c, pl.ds(nbase, NP_T)])

    @functools.partial(
        pl.kernel,
        out_type=jax.ShapeDtypeStruct((NC, GP, H), jnp.float32),
        mesh=mesh,
        compiler_params=pltpu.CompilerParams(use_tc_tiling_on_sc=False),
        scratch_types=[
            pltpu.VMEM((KN, 128), jnp.int32),
            pltpu.VMEM((KN, 128, H), jnp.float32),
            pltpu.VMEM((GP_T, H), jnp.float32),
            pltpu.VMEM_SHARED((GP, H), jnp.float32),
            pltpu.SemaphoreType.DMA,
            pltpu.SemaphoreType.DMA,
        ],
    )
    def _pool(tab_hbm, bat_hbm, out_hbm, batv, rows, zero_v, acc_sh,
              lsem, psem):
        c = lax.axis_index("c")
        s = lax.axis_index("s")
        wid = s * NC + c
        pltpu.sync_copy(bat_hbm.at[wid], batv)
        _fill_zeros(zero_v, GP_T)
        gbase = s * GP_T
        pltpu.sync_copy(zero_v, acc_sh.at[pl.ds(gbase, GP_T)])
        plsc.subcore_barrier()
        for j in range(KN):
            pltpu.async_copy(
                tab_hbm.at[pl.ds(wid * (KN * 128) + j * 128, 128)],
                rows.at[j], lsem)
        for j in range(KN):
            pltpu.make_async_copy(
                tab_hbm.at[pl.ds(wid * (KN * 128) + j * 128, 128)],
                rows.at[j], lsem).wait()
            pltpu.async_copy(rows.at[j], acc_sh.at[batv.at[j]], psem,
                             add=True)
        for j in range(KN):
            pltpu.make_async_copy(
                rows.at[j], acc_sh.at[batv.at[0]], psem).wait()
        plsc.subcore_barrier()
        pltpu.sync_copy(acc_sh.at[pl.ds(gbase, GP_T)],
                        out_hbm.at[c, pl.ds(gbase, GP_T)])

    return _deg_cnt, _agg, _pool


_RB = 2000


def _tc_h1(x_ref, dp_ref, w_ref, out_ref):
    d = dp_ref[...]
    dinv = lax.rsqrt(d[:, 0] + d[:, 1] + 1.0)
    h = jnp.dot(x_ref[...], w_ref[...], preferred_element_type=jnp.float32)
    out_ref[...] = h * dinv[:, None]


def _tc_mid(p0_ref, p1_ref, hp_ref, dp_ref, w_ref, b_ref, out_ref):
    d = dp_ref[...]
    dinv = lax.rsqrt(d[:, 0] + d[:, 1] + 1.0)
    agg = (p0_ref[...] + p1_ref[...] + hp_ref[...]) * dinv[:, None]
    a1 = jnp.maximum(agg + b_ref[...], 0.0)
    out_ref[...] = jnp.dot(a1, w_ref[...],
                           preferred_element_type=jnp.float32) * dinv[:, None]


def _tc_h2(q0_ref, q1_ref, hp_ref, dp_ref, b_ref, out_ref):
    d = dp_ref[...]
    dinv = lax.rsqrt(d[:, 0] + d[:, 1] + 1.0)
    out_ref[...] = (q0_ref[...] + q1_ref[...] + hp_ref[...]) * dinv[:, None] \
        + b_ref[...]


def _tc_head(s0_ref, s1_ref, cnt_ref, w1_ref, b1_ref, w2_ref, b2_ref, out_ref):
    cnt = jnp.maximum(cnt_ref[0] + cnt_ref[1], 1.0)
    pooled = (s0_ref[...] + s1_ref[...]) / cnt[:, None]
    t = jnp.maximum(
        jnp.dot(pooled, w1_ref[...], preferred_element_type=jnp.float32)
        + b1_ref[...], 0.0)
    out_ref[...] = jnp.dot(t, w2_ref[...],
                           preferred_element_type=jnp.float32) + b2_ref[...]


def _row_spec(width):
    return pl.BlockSpec((_RB, width), lambda i: (i, 0))


def _whole(shape):
    return pl.BlockSpec(shape, lambda *_: tuple(0 for _ in shape))


def kernel(x, edge_index, batch, W1, b1, W2, b2, fc1_W, fc1_b, fc2_W, fc2_b):
    _deg_cnt, _agg, _pool = _sc_kernels()

    src = edge_index[0].astype(jnp.int32)
    dst = edge_index[1].astype(jnp.int32)
    bat = batch.astype(jnp.int32)

    src3 = jnp.concatenate(
        [src, jnp.zeros((EPAD - E,), jnp.int32)]).reshape(NW, KE, 128)
    dst3 = jnp.concatenate(
        [dst, jnp.full((EPAD - E,), N, jnp.int32)]).reshape(NW, KE, 128)
    bat3 = jnp.concatenate(
        [bat, jnp.full((NPAD - N,), G, jnp.int32)]).reshape(NW, KN, 128)

    deg_p, cnt_p = _deg_cnt(dst3, bat3)
    dp = jnp.transpose(deg_p[:, :N, 0])
    cp = cnt_p[:, :G, 0]

    grid = (N // _RB,)
    h1p = pl.pallas_call(
        _tc_h1,
        grid=grid,
        in_specs=[_row_spec(D_IN), _row_spec(2), _whole((D_IN, H))],
        out_specs=_row_spec(H),
        out_shape=jax.ShapeDtypeStruct((N, H), jnp.float32),
    )(x, dp, W1)

    p = _agg(h1p, src3, dst3)
    h2p = pl.pallas_call(
        _tc_mid,
        grid=grid,
        in_specs=[_row_spec(H), _row_spec(H), _row_spec(H), _row_spec(2),
                  _whole((H, H)), _whole((1, H))],
        out_specs=_row_spec(H),
        out_shape=jax.ShapeDtypeStruct((N, H), jnp.float32),
    )(p[0, :N], p[1, :N], h1p, dp, W2, b1.reshape(1, H))

    q = _agg(h2p, src3, dst3)
    h2 = pl.pallas_call(
        _tc_h2,
        grid=grid,
        in_specs=[_row_spec(H), _row_spec(H), _row_spec(H), _row_spec(2),
                  _whole((1, H))],
        out_specs=_row_spec(H),
        out_shape=jax.ShapeDtypeStruct((N, H), jnp.float32),
    )(q[0, :N], q[1, :N], h2p, dp, b2.reshape(1, H))

    h2pad = jnp.concatenate(
        [h2, jnp.zeros((NPAD - N, H), jnp.float32)])
    sums = _pool(h2pad, bat3)

    out = pl.pallas_call(
        _tc_head,
        in_specs=[_whole((G, H)), _whole((G, H)), _whole((NC, G)),
                  _whole((H, H)), _whole((1, H)), _whole((H, 1)),
                  _whole((1, 1))],
        out_specs=_whole((G, 1)),
        out_shape=jax.ShapeDtypeStruct((G, 1), jnp.float32),
    )(sums[0, :G], sums[1, :G], cp, fc1_W, fc1_b.reshape(1, H),
      fc2_W, fc2_b.reshape(1, 1))
    return out[:, 0]


# speedup vs baseline: 3.7809x; 3.7809x over previous
import functools

import jax
import jax.numpy as jnp
from jax import lax
from jax.experimental import pallas as pl
from jax.experimental.pallas import tpu as pltpu
from jax.experimental.pallas import tpu_sc as plsc

N = 10000
E = 320000
G = 512
D_IN = 128
H = 32

NC = 2
NS = 16
NW = NC * NS
L = 16

KE = 80
EPAD = NW * KE * 128
NB = 10
AH = 5
NG = KE // NB

NP = 10240
NP_T = NP // NS
NPK = NP * H // 128
GP = 640
GP_T = GP // NS
GPK = GP * H // 128
KP = 5
PB = NP // NW // KP


def _perm_row(ref, j, ncols=128, dbl=False):
    for j0 in range(0, ncols, L):
        v = ref[j, j0:j0 + L]
        f = (v & jnp.int32(~2047)) + ((v & 511) << 2) + ((v >> 9) & 3)
        if dbl:
            f = f << 1
        ref[j, j0:j0 + L] = f


def _fill_const(ref, nrows, val):
    v16 = jnp.full((L,), val, jnp.float32)
    w = ref.shape[1]

    def body(i, _):
        for j0 in range(0, w, L):
            ref[i, j0:j0 + L] = v16
        return 0

    lax.fori_loop(0, nrows, body, 0)


@functools.cache
def _sc_kernels():
    mesh = plsc.VectorSubcoreMesh(core_axis_name="c", subcore_axis_name="s")

    @functools.partial(
        pl.kernel,
        out_type=[
            jax.ShapeDtypeStruct((NC, 2 * NP, L), jnp.float32),
            jax.ShapeDtypeStruct((NC, 2 * GP, L), jnp.float32),
        ],
        mesh=mesh,
        compiler_params=pltpu.CompilerParams(use_tc_tiling_on_sc=False),
        scratch_types=[
            pltpu.VMEM((KE, 128), jnp.int32),
            pltpu.VMEM((KP, PB), jnp.int32),
            pltpu.VMEM((128, L), jnp.float32),
            pltpu.VMEM((128, L), jnp.float32),
            pltpu.VMEM_SHARED((2 * NP, L), jnp.float32),
            pltpu.VMEM_SHARED((2 * GP, L), jnp.float32),
            pltpu.SemaphoreType.DMA,
            pltpu.SemaphoreType.DMA,
        ],
    )
    def _deg_cnt(dst_hbm, bat_hbm, deg_out, cnt_out,
                 dstv, batv, ones_v, zero_v, deg_sh, cnt_sh, dsem, bsem):
        c = lax.axis_index("c")
        s = lax.axis_index("s")
        wid = s * NC + c
        sdesc = pltpu.async_copy(dst_hbm.at[wid], dstv, dsem)
        bdesc = pltpu.async_copy(bat_hbm.at[wid], batv, bsem)
        _fill_const(ones_v, 128, 1.0)
        _fill_const(zero_v, 128, 0.0)
        nbase = s * (2 * NP_T)
        for j in range(2 * NP_T // 128):
            pltpu.sync_copy(zero_v, deg_sh.at[pl.ds(nbase + j * 128, 128)])
        gbase = s * (2 * GP_T)
        pltpu.sync_copy(zero_v.at[pl.ds(0, 2 * GP_T)],
                        cnt_sh.at[pl.ds(gbase, 2 * GP_T)])
        sdesc.wait()
        bdesc.wait()

        def _dbl_rows(ref, nrows, ncols):
            def body(i, _):
                for j0 in range(0, ncols, L):
                    ref[i, j0:j0 + L] = ref[i, j0:j0 + L] * 2
                return 0
            lax.fori_loop(0, nrows, body, 0)

        _dbl_rows(batv, KP, PB)
        plsc.subcore_barrier()

        def d_start(j):
            _perm_row(dstv, j, dbl=True)
            pltpu.async_copy(ones_v, deg_sh.at[dstv.at[j]], dsem, add=True)

        def d_wait():
            pltpu.make_async_copy(ones_v, deg_sh.at[dstv.at[0]], dsem).wait()

        for j in range(KP):
            pltpu.async_copy(ones_v.at[pl.ds(0, PB)],
                             cnt_sh.at[batv.at[j]], bsem, add=True)
        for b in range(NB):
            d_start(b)

        def grp(g, _):
            for b in range(NB):
                d_start(g * NB + b)
                d_wait()
            return 0

        lax.fori_loop(1, NG, grp, 0)
        for b in range(NB):
            d_wait()
        for j in range(KP):
            pltpu.make_async_copy(ones_v.at[pl.ds(0, PB)],
                                  cnt_sh.at[batv.at[0]], bsem).wait()
        plsc.subcore_barrier()
        pltpu.sync_copy(deg_sh.at[pl.ds(nbase, 2 * NP_T)],
                        deg_out.at[c, pl.ds(nbase, 2 * NP_T)])
        pltpu.sync_copy(cnt_sh.at[pl.ds(gbase, 2 * GP_T)],
                        cnt_out.at[c, pl.ds(gbase, 2 * GP_T)])

    @functools.partial(
        pl.kernel,
        out_type=jax.ShapeDtypeStruct((NC, NP, H), jnp.float32),
        mesh=mesh,
        compiler_params=pltpu.CompilerParams(use_tc_tiling_on_sc=False),
        scratch_types=[
            pltpu.VMEM((KE, 128), jnp.int32),
            pltpu.VMEM((KE, 128), jnp.int32),
            pltpu.VMEM((NB, 128, H), jnp.float32),
            pltpu.VMEM((128, H), jnp.float32),
            pltpu.VMEM_SHARED((NP, H), jnp.float32),
            pltpu.SemaphoreType.DMA((NB,)),
            pltpu.SemaphoreType.DMA((NB,)),
        ],
    )
    def _agg(tab_hbm, src_hbm, dst_hbm, out_hbm,
             srcv, dstv, rows, zero_v, acc_sh, gsem, ssem):
        c = lax.axis_index("c")
        s = lax.axis_index("s")
        wid = s * NC + c
        sdesc = pltpu.async_copy(src_hbm.at[wid], srcv, gsem.at[0])
        ddesc = pltpu.async_copy(dst_hbm.at[wid], dstv, gsem.at[1])
        _fill_const(zero_v, 128, 0.0)
        nbase = s * NP_T
        for j in range(NP_T // 128):
            pltpu.sync_copy(zero_v, acc_sh.at[pl.ds(nbase + j * 128, 128)])
        sdesc.wait()
        ddesc.wait()

        def g_start(j, b):
            _perm_row(srcv, j)
            pltpu.async_copy(tab_hbm.at[srcv.at[j]], rows.at[b], gsem.at[b])

        def g_wait(j, b):
            pltpu.make_async_copy(
                tab_hbm.at[srcv.at[j]], rows.at[b], gsem.at[b]).wait()

        def s_start(j, b):
            _perm_row(dstv, j)
            pltpu.async_copy(rows.at[b], acc_sh.at[dstv.at[j]], ssem.at[b],
                             add=True)

        def s_wait(j, b):
            pltpu.make_async_copy(
                rows.at[b], acc_sh.at[dstv.at[j]], ssem.at[b]).wait()

        for b in range(AH):
            g_start(b, b)
        plsc.subcore_barrier()
        for b in range(NB):
            j = b
            g_wait(j, b)
            s_start(j, b)
            bn = (b + AH) % NB
            if b >= AH:
                s_wait(j + AH - NB, bn)
            g_start(j + AH, bn)

        def grp(g, _):
            for b in range(NB):
                j = g * NB + b
                g_wait(j, b)
                s_start(j, b)
                bn = (b + AH) % NB
                s_wait(j + AH - NB, bn)
                g_start(j + AH, bn)
            return 0

        lax.fori_loop(1, NG - 1, grp, 0)
        for b in range(NB):
            j = (NG - 1) * NB + b
            g_wait(j, b)
            s_start(j, b)
            if b < AH:
                bn = (b + AH) % NB
                s_wait(j + AH - NB, bn)
                g_start(j + AH, bn)
        for b in range(NB):
            s_wait(KE - NB + b, b)
        plsc.subcore_barrier()
        pltpu.sync_copy(acc_sh.at[pl.ds(nbase, NP_T)],
                        out_hbm.at[c, pl.ds(nbase, NP_T)])

    @functools.partial(
        pl.kernel,
        out_type=jax.ShapeDtypeStruct((NC, GP, H), jnp.float32),
        mesh=mesh,
        compiler_params=pltpu.CompilerParams(use_tc_tiling_on_sc=False),
        scratch_types=[
            pltpu.VMEM((KP, PB), jnp.int32),
            pltpu.VMEM((KP, PB, H), jnp.float32),
            pltpu.VMEM((GP_T, H), jnp.float32),
            pltpu.VMEM_SHARED((GP, H), jnp.float32),
            pltpu.SemaphoreType.DMA,
            pltpu.SemaphoreType.DMA,
        ],
    )
    def _pool(tab_hbm, bat_hbm, out_hbm, batv, rows, zero_v, acc_sh,
              lsem, psem):
        c = lax.axis_index("c")
        s = lax.axis_index("s")
        wid = s * NC + c
        pltpu.sync_copy(bat_hbm.at[wid], batv)
        _fill_const(zero_v, GP_T, 0.0)
        gbase = s * GP_T
        pltpu.sync_copy(zero_v, acc_sh.at[pl.ds(gbase, GP_T)])
        plsc.subcore_barrier()
        for j in range(KP):
            pltpu.async_copy(
                tab_hbm.at[pl.ds(wid * (KP * PB) + j * PB, PB)],
                rows.at[j], lsem)
        for j in range(KP):
            pltpu.make_async_copy(
                tab_hbm.at[pl.ds(wid * (KP * PB) + j * PB, PB)],
                rows.at[j], lsem).wait()
            pltpu.async_copy(rows.at[j], acc_sh.at[batv.at[j]], psem,
                             add=True)
        for j in range(KP):
            pltpu.make_async_copy(
                rows.at[j], acc_sh.at[batv.at[0]], psem).wait()
        plsc.subcore_barrier()
        pltpu.sync_copy(acc_sh.at[pl.ds(gbase, GP_T)],
                        out_hbm.at[c, pl.ds(gbase, GP_T)])

    return _deg_cnt, _agg, _pool


_RB = 2048
_RP = _RB * H // 128


def _pack(h):
    q = h.shape[0] // 4
    return jnp.concatenate([h[k * q:(k + 1) * q] for k in range(4)], axis=1)


def _unpack(hp):
    return jnp.concatenate([hp[:, k * H:(k + 1) * H] for k in range(4)],
                           axis=0)


def _tc_mm1(x_ref, w_ref, out_ref):
    h = jnp.dot(x_ref[...], w_ref[...], preferred_element_type=jnp.float32)
    out_ref[...] = _pack(h)


def _drep(d):
    return d + jnp.concatenate([d[:, -16:], d[:, :-16]], axis=1)


def _tc_scale(hp_ref, dp0_ref, dp1_ref, out_ref):
    dinv = lax.rsqrt(_drep(dp0_ref[0] + dp1_ref[0]) + 1.0)
    out_ref[...] = hp_ref[...] * dinv


def _tc_mid(p0_ref, p1_ref, hp_ref, dp0_ref, dp1_ref, w_ref, b_ref, out_ref):
    dinv = lax.rsqrt(_drep(dp0_ref[0] + dp1_ref[0]) + 1.0)
    a1 = jnp.maximum(
        (p0_ref[0] + p1_ref[0] + hp_ref[...]) * dinv + b_ref[...], 0.0)
    h2 = jnp.dot(_unpack(a1), w_ref[...],
                 preferred_element_type=jnp.float32)
    out_ref[...] = _pack(h2) * dinv


def _tc_h2(q0_ref, q1_ref, hp_ref, dp0_ref, dp1_ref, b_ref, out_ref):
    dinv = lax.rsqrt(_drep(dp0_ref[0] + dp1_ref[0]) + 1.0)
    out_ref[...] = (q0_ref[0] + q1_ref[0] + hp_ref[...]) * dinv + b_ref[...]


def _tc_head(s0_ref, s1_ref, c0_ref, c1_ref, w1_ref, b1_ref, w2_ref, b2_ref,
             out_ref):
    cnt = jnp.maximum(_drep(c0_ref[0] + c1_ref[0]), 1.0)
    pooled = _unpack((s0_ref[0] + s1_ref[0]) / cnt)
    t = jnp.maximum(
        jnp.dot(pooled, w1_ref[...], preferred_element_type=jnp.float32)
        + b1_ref[...], 0.0)
    out_ref[...] = jnp.dot(t, w2_ref[...],
                           preferred_element_type=jnp.float32) + b2_ref[...]


def _prow(part):
    return pl.BlockSpec((1, _RP, 128), lambda i, _p=part: (_p, i, 0))


def _row(shape):
    return pl.BlockSpec(shape, lambda i: (i, 0))


def kernel(x, edge_index, batch, W1, b1, W2, b2, fc1_W, fc1_b, fc2_W, fc2_b):
    _deg_cnt, _agg, _pool = _sc_kernels()
    f32 = jnp.float32

    src = edge_index[0].astype(jnp.int32)
    dst = edge_index[1].astype(jnp.int32)
    bat = batch.astype(jnp.int32)

    def perm(v):
        blk = v // _RB
        return blk * _RB + (v % _RP) * 4 + (v % _RB) // _RP

    e_pad = EPAD - E
    n_pad = NP - N
    ei2 = edge_index.astype(jnp.int32).reshape(2, E // 128, 128)
    pad_src = (jnp.arange(e_pad, dtype=jnp.int32) % N
               ).reshape(e_pad // 128, 128)
    pad_dst = (N + jnp.arange(e_pad, dtype=jnp.int32) % n_pad
               ).reshape(e_pad // 128, 128)
    src3 = jnp.concatenate([ei2[0], pad_src]).reshape(NW, KE, 128)
    dst3 = jnp.concatenate([ei2[1], pad_dst]).reshape(NW, KE, 128)
    bat_pad = jnp.concatenate(
        [bat, G + jnp.arange(n_pad, dtype=jnp.int32) % (GP - G)])
    bat3 = bat_pad.reshape(NP // _RB, 4, _RP).transpose(0, 2, 1).reshape(
        NW, KP, PB)

    deg_p, cnt_p = _deg_cnt(dst3, bat3)
    dpp = jnp.reshape(deg_p, (NC, NPK, 128))
    cpp = jnp.reshape(cnt_p, (NC, GPK, 128))

    grid = (NP // _RB,)
    h1m = pl.pallas_call(
        _tc_mm1,
        grid=grid,
        in_specs=[_row((_RB, D_IN)),
                  pl.BlockSpec((D_IN, H), lambda i: (0, 0))],
        out_specs=_row((_RP, 128)),
        out_shape=jax.ShapeDtypeStruct((NPK, 128), f32),
    )(x, W1)
    h1p = pl.pallas_call(
        _tc_scale,
        grid=grid,
        in_specs=[_row((_RP, 128)), _prow(0), _prow(1)],
        out_specs=_row((_RP, 128)),
        out_shape=jax.ShapeDtypeStruct((NPK, 128), f32),
    )(h1m, dpp, dpp)

    p = _agg(jnp.reshape(h1p, (NP, H)), src3, dst3)
    pp = jnp.reshape(p, (NC, NPK, 128))
    b1p = jnp.tile(b1, 4).reshape(1, 128)
    h2p = pl.pallas_call(
        _tc_mid,
        grid=grid,
        in_specs=[_prow(0), _prow(1), _row((_RP, 128)), _prow(0), _prow(1),
                  pl.BlockSpec((H, H), lambda i: (0, 0)),
                  pl.BlockSpec((1, 128), lambda i: (0, 0))],
        out_specs=_row((_RP, 128)),
        out_shape=jax.ShapeDtypeStruct((NPK, 128), f32),
    )(pp, pp, h1p, dpp, dpp, W2, b1p)

    q = _agg(jnp.reshape(h2p, (NP, H)), src3, dst3)
    qp = jnp.reshape(q, (NC, NPK, 128))
    b2p = jnp.tile(b2, 4).reshape(1, 128)
    h2 = pl.pallas_call(
        _tc_h2,
        grid=grid,
        in_specs=[_prow(0), _prow(1), _row((_RP, 128)), _prow(0), _prow(1),
                  pl.BlockSpec((1, 128), lambda i: (0, 0))],
        out_specs=_row((_RP, 128)),
        out_shape=jax.ShapeDtypeStruct((NPK, 128), f32),
    )(qp, qp, h2p, dpp, dpp, b2p)

    sums = _pool(jnp.reshape(h2, (NP, H)), bat3)
    sp = jnp.reshape(sums, (NC, GPK, 128))

    gpk = G * H // 128
    out = pl.pallas_call(
        _tc_head,
        grid=(1,),
        in_specs=[
            pl.BlockSpec((1, gpk, 128), lambda i: (0, 0, 0)),
            pl.BlockSpec((1, gpk, 128), lambda i: (1, 0, 0)),
            pl.BlockSpec((1, gpk, 128), lambda i: (0, 0, 0)),
            pl.BlockSpec((1, gpk, 128), lambda i: (1, 0, 0)),
            pl.BlockSpec((H, H), lambda i: (0, 0)),
            pl.BlockSpec((1, H), lambda i: (0, 0)),
            pl.BlockSpec((H, 1), lambda i: (0, 0)),
            pl.BlockSpec((1, 1), lambda i: (0, 0)),
        ],
        out_specs=pl.BlockSpec((G, 1), lambda i: (0, 0)),
        out_shape=jax.ShapeDtypeStruct((G, 1), f32),
    )(sp, sp, cpp, cpp, fc1_W, fc1_b.reshape(1, H), fc2_W,
      fc2_b.reshape(1, 1))
    return out[:, 0].reshape(4, G // 4).T.reshape(G)
